# Initial kernel scaffold; baseline (speedup 1.0000x reference)
import jax
import jax.numpy as jnp
from jax.experimental import pallas as pl


def kernel(edge_index, edge_weight, embeds):
    raise NotImplementedError("write your pallas kernel here")


# speedup vs baseline: 3.3427x; 3.3427x over previous
import functools

import jax
import jax.numpy as jnp
from jax import lax
from jax.experimental import pallas as pl
from jax.experimental.pallas import tpu as pltpu
from jax.experimental.pallas import tpu_sc as plsc

_N = 10000
_D = 128
_E = 320000
_NC = 2
_NS = 16
_NW = _NC * _NS
_K = 128
_CPW = 79
_EPAD = _NW * _CPW * _K
_RPS = 624
_RTAIL = _N - _RPS * _NS


def _layernorm(x):
    def body(x_ref, o_ref):
        v = x_ref[...]
        m = jnp.mean(v, axis=-1, keepdims=True)
        d = v - m
        var = jnp.mean(d * d, axis=-1, keepdims=True)
        o_ref[...] = d * lax.rsqrt(var + 1e-5)

    return pl.pallas_call(
        body,
        out_shape=jax.ShapeDtypeStruct((_N, _D), jnp.float32),
        grid=(10,),
        in_specs=[pl.BlockSpec((_N // 10, _D), lambda i: (i, 0))],
        out_specs=pl.BlockSpec((_N // 10, _D), lambda i: (i, 0)),
    )(x)


def _add2(a, b):
    def body(a_ref, b_ref, o_ref):
        o_ref[...] = a_ref[...] + b_ref[...]

    return pl.pallas_call(
        body,
        out_shape=jax.ShapeDtypeStruct((_N, _D), jnp.float32),
        grid=(10,),
        in_specs=[
            pl.BlockSpec((_N // 10, _D), lambda i: (i, 0)),
            pl.BlockSpec((_N // 10, _D), lambda i: (i, 0)),
        ],
        out_specs=pl.BlockSpec((_N // 10, _D), lambda i: (i, 0)),
    )(a, b)


def _spmm_body(src_h, dst_h, w_h, x_h, init_h, out_h,
               src_v, dst_v, w_v, rows_v, acc, sem):
    c = lax.axis_index("c")
    s = lax.axis_index("s")
    r0 = s * _RPS
    pltpu.sync_copy(init_h.at[c, pl.ds(r0, _RPS)], acc.at[pl.ds(r0, _RPS)])

    @pl.when(s == 0)
    def _():
        pltpu.sync_copy(init_h.at[c, pl.ds(_RPS * _NS, _RTAIL)],
                        acc.at[pl.ds(_RPS * _NS, _RTAIL)])

    plsc.subcore_barrier()

    base_chunk = (c * _NS + s) * _CPW

    def chunk_body(g, carry):
        eb = (base_chunk + g) * _K
        pltpu.sync_copy(src_h.at[pl.ds(eb, _K)], src_v)
        pltpu.sync_copy(dst_h.at[pl.ds(eb, _K)], dst_v)
        pltpu.sync_copy(w_h.at[pl.ds(eb, _K)], w_v)
        pltpu.async_copy(x_h.at[src_v], rows_v, sem).wait()

        def scale_body(g16, carry2):
            wg = w_v[pl.ds(g16 * 16, 16)]
            for e16 in range(16):
                e = g16 * 16 + e16
                wb = wg.at[jnp.full((16,), e16, jnp.int32)].get(
                    mode="promise_in_bounds")
                for r in range(_D // 16):
                    rows_v[e, pl.ds(r * 16, 16)] = (
                        rows_v[e, pl.ds(r * 16, 16)] * wb)
            return carry2

        lax.fori_loop(0, _K // 16, scale_body, 0)
        pltpu.sync_copy(rows_v, acc.at[dst_v], add=True)
        return carry

    lax.fori_loop(0, _CPW, chunk_body, 0)
    plsc.subcore_barrier()
    pltpu.sync_copy(acc.at[pl.ds(r0, _RPS)], out_h.at[c, pl.ds(r0, _RPS)])

    @pl.when(s == 0)
    def _():
        pltpu.sync_copy(acc.at[pl.ds(_RPS * _NS, _RTAIL)],
                        out_h.at[c, pl.ds(_RPS * _NS, _RTAIL)])


def _spmm_sc(src, dst, w, x, init):
    mesh = plsc.VectorSubcoreMesh(core_axis_name="c", subcore_axis_name="s")
    f = functools.partial(
        pl.kernel,
        out_type=jax.ShapeDtypeStruct((_NC, _N, _D), jnp.float32),
        mesh=mesh,
        scratch_types=[
            pltpu.VMEM((_K,), jnp.int32),
            pltpu.VMEM((_K,), jnp.int32),
            pltpu.VMEM((_K,), jnp.float32),
            pltpu.VMEM((_K, _D), jnp.float32),
            pltpu.VMEM_SHARED((_N, _D), jnp.float32),
            pltpu.SemaphoreType.DMA,
        ],
    )(_spmm_body)
    return f(src, dst, w, x, init)


def kernel(edge_index, edge_weight, embeds):
    src = edge_index[1]
    dst = edge_index[0]
    pad = _EPAD - _E
    src_p = jnp.pad(src, (0, pad))
    dst_p = jnp.pad(dst, (0, pad))
    w_p = jnp.pad(edge_weight, (0, pad))

    x = _layernorm(embeds)
    zeros2 = jnp.zeros((_NC, _N, _D), jnp.float32)
    p = _spmm_sc(src_p, dst_p, w_p, x, zeros2)
    y1 = _add2(p[0], p[1])
    init2 = jnp.concatenate([y1[None], jnp.zeros((1, _N, _D), jnp.float32)], axis=0)
    q = _spmm_sc(src_p, dst_p, w_p, y1, init2)
    return _add2(q[0], q[1])
